# baseline (speedup 1.0000x reference)
import functools

import jax
import jax.numpy as jnp
from jax import lax
from jax.experimental import pallas as pl
from jax.experimental.pallas import tpu as pltpu
from jax.experimental.pallas import tpu_sc as plsc

_B = 16
_N = 5000
_M = 8192
_S = 8
_R = 64 * _S
_C = 128

_NW = 32
_CH = (_B * _M) // _NW


def _sort_body(prob_ref, scores_ref, gidx_ref):
    shp = (1, _R, _C)
    r_iota = jax.lax.broadcasted_iota(jnp.int32, shp, 1)
    c_iota = jax.lax.broadcasted_iota(jnp.int32, shp, 2)
    rloc = r_iota & 63
    lin = rloc * _C + c_iota

    prob = prob_ref[...]
    key = jax.lax.bitcast_convert_type(prob, jnp.int32)
    idx = lin

    def cmpx(arrs, asc, is_hi, partners):
        flip = jnp.logical_xor(is_hi, asc)
        k, i = arrs
        kp, ip = partners
        before = (k > kp) | ((k == kp) & (i < ip))
        take_mine = jnp.logical_xor(before, flip)
        return [jnp.where(take_mine, a, p) for a, p in zip(arrs, partners)]

    def lane_pass(arrs, asc, d):
        is_hi = (c_iota & d) != 0
        partners = [
            jnp.where(is_hi, pltpu.roll(a, d, 2), pltpu.roll(a, _C - d, 2))
            for a in arrs
        ]
        return cmpx(arrs, asc, is_hi, partners)

    def row_pass(arrs, asc, dr):
        is_hi = (r_iota & dr) != 0
        partners = [
            jnp.where(is_hi, pltpu.roll(a, dr, 1), pltpu.roll(a, _R - dr, 1))
            for a in arrs
        ]
        return cmpx(arrs, asc, is_hi, partners)

    arrs = [key, idx]
    for size in [2 << s for s in range(13)]:
        asc = (lin & size) != 0
        d = size // 2
        while d >= 128:
            arrs = row_pass(arrs, asc, d // 128)
            d //= 2
        while d >= 1:
            arrs = lane_pass(arrs, asc, d)
            d //= 2

    img = pl.program_id(0) * _S + (r_iota >> 6)
    sidx = arrs[1]
    scores_ref[...] = jax.lax.bitcast_convert_type(arrs[0], jnp.float32)
    gidx_ref[...] = jnp.where(sidx < _N, img * _N + sidx, 0)


def _gather_body(idx_hbm, t0, t1, t2, t3, sc_hbm, o0, o1, o2, o3,
                 idx_v, b0, b1, b2, b3, scv, sem):
    wid = lax.axis_index("s") * 2 + lax.axis_index("c")
    base = wid * _CH
    pltpu.sync_copy(idx_hbm.at[pl.ds(base, _CH)], idx_v)
    pltpu.sync_copy(sc_hbm.at[pl.ds(wid * 32, 32)], scv)
    cps = [
        pltpu.async_copy(t.at[idx_v], b, sem)
        for t, b in ((t0, b0), (t1, b1), (t2, b2), (t3, b3))
    ]
    for c in cps:
        c.wait()

    iwv = scv[pl.ds(0, 16)]
    ihv = scv[pl.ds(16, 16)]

    def step(v, carry):
        off = v * 16
        cxv = b0[pl.ds(off, 16)]
        cyv = b1[pl.ds(off, 16)]
        wv = b2[pl.ds(off, 16)]
        hv = b3[pl.ds(off, 16)]
        b0[pl.ds(off, 16)] = (cxv - 0.5 * wv) * iwv
        b1[pl.ds(off, 16)] = (cyv - 0.5 * hv) * ihv
        b2[pl.ds(off, 16)] = (cxv + 0.5 * wv) * iwv
        b3[pl.ds(off, 16)] = (cyv + 0.5 * hv) * ihv
        return carry

    lax.fori_loop(0, _CH // 16, step, 0, unroll=4)

    for b, o in ((b0, o0), (b1, o1), (b2, o2), (b3, o3)):
        pltpu.sync_copy(b, o.at[pl.ds(base, _CH)])


@functools.partial(jax.jit, static_argnames=())
def kernel(pred_logits, pred_boxes, target_sizes):
    B, N, C = pred_logits.shape
    assert (B, N, C) == (_B, _N, 1)
    nblk = B // _S

    prob = jax.nn.sigmoid(pred_logits.reshape(B, N))
    pad = _M - N
    prob_p = jnp.pad(prob, ((0, 0), (0, pad)), constant_values=-1.0).reshape(nblk, _R, _C)

    blk = pl.BlockSpec((1, _R, _C), lambda b: (b, 0, 0))
    out_shape = [
        jax.ShapeDtypeStruct((nblk, _R, _C), jnp.float32),
        jax.ShapeDtypeStruct((nblk, _R, _C), jnp.int32),
    ]
    scores, gidx = pl.pallas_call(
        _sort_body,
        grid=(nblk,),
        in_specs=[blk],
        out_specs=[blk] * 2,
        out_shape=out_shape,
    )(prob_p)

    t0 = pred_boxes[:, :, 0].reshape(B * N)
    t1 = pred_boxes[:, :, 1].reshape(B * N)
    t2 = pred_boxes[:, :, 2].reshape(B * N)
    t3 = pred_boxes[:, :, 3].reshape(B * N)
    img_h = target_sizes[:, 0].astype(jnp.float32)
    img_w = target_sizes[:, 1].astype(jnp.float32)
    tile_img = jnp.arange(_NW, dtype=jnp.int32) // 2
    scflat = jnp.stack(
        [jnp.broadcast_to(img_w[tile_img][:, None], (_NW, 16)),
         jnp.broadcast_to(img_h[tile_img][:, None], (_NW, 16))],
        axis=1).reshape(_NW * 32)

    flat = (_B * _M,)
    mesh = plsc.VectorSubcoreMesh(core_axis_name="c", subcore_axis_name="s")
    gathered = pl.kernel(
        _gather_body,
        mesh=mesh,
        out_type=[jax.ShapeDtypeStruct(flat, jnp.float32)] * 4,
        scratch_types=[
            pltpu.VMEM((_CH,), jnp.int32),
            pltpu.VMEM((_CH,), jnp.float32),
            pltpu.VMEM((_CH,), jnp.float32),
            pltpu.VMEM((_CH,), jnp.float32),
            pltpu.VMEM((_CH,), jnp.float32),
            pltpu.VMEM((32,), jnp.float32),
            pltpu.SemaphoreType.DMA,
        ],
    )(gidx.reshape(flat), t0, t1, t2, t3, scflat)

    scores = scores.reshape(B, _M)[:, :_N]
    boxes = jnp.stack(
        [g.reshape(B, _M)[:, :_N] for g in gathered], axis=-1)
    labels = jnp.full((B, N), 1, dtype=jnp.int32)
    return scores, labels, boxes


# speedup vs baseline: 7.8940x; 7.8940x over previous
import functools

import jax
import jax.numpy as jnp
from jax import lax
from jax.experimental import pallas as pl
from jax.experimental.pallas import tpu as pltpu
from jax.experimental.pallas import tpu_sc as plsc

_B = 16
_N = 5000
_M = 8192
_S = 8
_R = 64 * _S
_C = 128

_NW = 32
_CH = (_B * _M) // _NW


def _sort_body(prob_ref, cx_ref, cy_ref, w_ref, h_ref, sw_ref, sh_ref,
               scores_ref, gidx_ref, x1_ref, y1_ref, x2_ref, y2_ref):
    shp = (1, _R, _C)
    r_iota = jax.lax.broadcasted_iota(jnp.int32, shp, 1)
    c_iota = jax.lax.broadcasted_iota(jnp.int32, shp, 2)
    rloc = r_iota & 63
    lin = rloc * _C + c_iota

    prob = prob_ref[...]
    key = jax.lax.bitcast_convert_type(prob, jnp.int32)
    idx = lin

    iw = sw_ref[...]
    ih = sh_ref[...]
    cx = cx_ref[...]
    cy = cy_ref[...]
    w = w_ref[...]
    h = h_ref[...]
    x1_ref[...] = (cx - 0.5 * w) * iw
    y1_ref[...] = (cy - 0.5 * h) * ih
    x2_ref[...] = (cx + 0.5 * w) * iw
    y2_ref[...] = (cy + 0.5 * h) * ih

    def cmpx(arrs, asc, is_hi, partners):
        flip = jnp.logical_xor(is_hi, asc)
        k, i = arrs
        kp, ip = partners
        before = (k > kp) | ((k == kp) & (i < ip))
        take_mine = jnp.logical_xor(before, flip)
        return [jnp.where(take_mine, a, p) for a, p in zip(arrs, partners)]

    def lane_pass(arrs, asc, d):
        is_hi = (c_iota & d) != 0
        partners = [
            jnp.where(is_hi, pltpu.roll(a, d, 2), pltpu.roll(a, _C - d, 2))
            for a in arrs
        ]
        return cmpx(arrs, asc, is_hi, partners)

    def row_pass(arrs, asc, dr):
        is_hi = (r_iota & dr) != 0
        partners = [
            jnp.where(is_hi, pltpu.roll(a, dr, 1), pltpu.roll(a, _R - dr, 1))
            for a in arrs
        ]
        return cmpx(arrs, asc, is_hi, partners)

    arrs = [key, idx]
    for size in [2 << s for s in range(13)]:
        asc = (lin & size) != 0
        d = size // 2
        while d >= 128:
            arrs = row_pass(arrs, asc, d // 128)
            d //= 2
        while d >= 1:
            arrs = lane_pass(arrs, asc, d)
            d //= 2

    base = pl.program_id(0) * (_S * _M) + (r_iota >> 6) * _M
    scores_ref[...] = jax.lax.bitcast_convert_type(arrs[0], jnp.float32)
    gidx_ref[...] = arrs[1] + base


def _gather_body(idx_hbm, t0, t1, t2, t3, o0, o1, o2, o3,
                 idx_v, b0, b1, b2, b3, sem):
    wid = lax.axis_index("s") * 2 + lax.axis_index("c")
    base = wid * _CH
    pltpu.sync_copy(idx_hbm.at[pl.ds(base, _CH)], idx_v)
    cps = [
        pltpu.async_copy(t.at[idx_v], b, sem)
        for t, b in ((t0, b0), (t1, b1), (t2, b2), (t3, b3))
    ]
    for c in cps:
        c.wait()
    for b, o in ((b0, o0), (b1, o1), (b2, o2), (b3, o3)):
        pltpu.sync_copy(b, o.at[pl.ds(base, _CH)])


@functools.partial(jax.jit, static_argnames=())
def kernel(pred_logits, pred_boxes, target_sizes):
    B, N, C = pred_logits.shape
    assert (B, N, C) == (_B, _N, 1)
    nblk = B // _S

    prob = jax.nn.sigmoid(pred_logits.reshape(B, N))
    pad = _M - N
    prob_p = jnp.pad(prob, ((0, 0), (0, pad)), constant_values=-1.0).reshape(nblk, _R, _C)

    cx = jnp.pad(pred_boxes[:, :, 0], ((0, 0), (0, pad))).reshape(nblk, _R, _C)
    cy = jnp.pad(pred_boxes[:, :, 1], ((0, 0), (0, pad))).reshape(nblk, _R, _C)
    w = jnp.pad(pred_boxes[:, :, 2], ((0, 0), (0, pad))).reshape(nblk, _R, _C)
    h = jnp.pad(pred_boxes[:, :, 3], ((0, 0), (0, pad))).reshape(nblk, _R, _C)

    img_h = target_sizes[:, 0].astype(jnp.float32)
    img_w = target_sizes[:, 1].astype(jnp.float32)
    sw = jnp.broadcast_to(img_w[:, None, None], (B, 64, _C)).reshape(nblk, _R, _C)
    sh = jnp.broadcast_to(img_h[:, None, None], (B, 64, _C)).reshape(nblk, _R, _C)

    blk = pl.BlockSpec((1, _R, _C), lambda b: (b, 0, 0))
    out_shape = [
        jax.ShapeDtypeStruct((nblk, _R, _C), jnp.float32),
        jax.ShapeDtypeStruct((nblk, _R, _C), jnp.int32),
        jax.ShapeDtypeStruct((nblk, _R, _C), jnp.float32),
        jax.ShapeDtypeStruct((nblk, _R, _C), jnp.float32),
        jax.ShapeDtypeStruct((nblk, _R, _C), jnp.float32),
        jax.ShapeDtypeStruct((nblk, _R, _C), jnp.float32),
    ]
    scores, gidx, x1, y1, x2, y2 = pl.pallas_call(
        _sort_body,
        grid=(nblk,),
        in_specs=[blk] * 7,
        out_specs=[blk] * 6,
        out_shape=out_shape,
    )(prob_p, cx, cy, w, h, sw, sh)

    flat = (_B * _M,)
    mesh = plsc.VectorSubcoreMesh(core_axis_name="c", subcore_axis_name="s")
    gathered = pl.kernel(
        _gather_body,
        mesh=mesh,
        out_type=[jax.ShapeDtypeStruct(flat, jnp.float32)] * 4,
        scratch_types=[
            pltpu.VMEM((_CH,), jnp.int32),
            pltpu.VMEM((_CH,), jnp.float32),
            pltpu.VMEM((_CH,), jnp.float32),
            pltpu.VMEM((_CH,), jnp.float32),
            pltpu.VMEM((_CH,), jnp.float32),
            pltpu.SemaphoreType.DMA,
        ],
    )(gidx.reshape(flat), x1.reshape(flat), y1.reshape(flat),
      x2.reshape(flat), y2.reshape(flat))

    scores = scores.reshape(B, _M)[:, :_N]
    boxes = jnp.stack(
        [g.reshape(B, _M)[:, :_N] for g in gathered], axis=-1)
    labels = jnp.full((B, N), 1, dtype=jnp.int32)
    return scores, labels, boxes
